# baseline (speedup 1.0000x reference)
import functools

import jax
import jax.numpy as jnp
from jax import lax
from jax.experimental import pallas as pl
from jax.experimental.pallas import tpu as pltpu
from jax.experimental.pallas import tpu_sc as plsc

_SC_ROWS = 2048
_SC_CHUNK = 32
_TC_CHUNK = 2048
_NBUF = 4
_PREFETCH = 3


def _tc_copy_head(table, head_rows):
    max_len, emb_dim = table.shape
    nch = head_rows // _TC_CHUNK

    def body(in_hbm, out_hbm, buf, *sems):
        sin = sems[:_NBUF]
        sout = sems[_NBUF:]

        def cin(i):
            return pltpu.make_async_copy(
                in_hbm.at[pl.ds(i * _TC_CHUNK, _TC_CHUNK)],
                buf.at[i % _NBUF],
                sin[i % _NBUF],
            )

        def cout(i):
            return pltpu.make_async_copy(
                buf.at[i % _NBUF],
                out_hbm.at[pl.ds(i * _TC_CHUNK, _TC_CHUNK)],
                sout[i % _NBUF],
            )

        for i in range(min(_PREFETCH, nch)):
            cin(i).start()
        for i in range(nch):
            cin(i).wait()
            cout(i).start()
            j = i + _PREFETCH
            if j < nch:
                if j >= _NBUF:
                    cout(j - _NBUF).wait()
                cin(j).start()
        for i in range(max(nch - _NBUF, 0), nch):
            cout(i).wait()

    return pl.pallas_call(
        body,
        in_specs=[pl.BlockSpec(memory_space=pltpu.MemorySpace.HBM)],
        out_specs=pl.BlockSpec(memory_space=pltpu.MemorySpace.HBM),
        out_shape=jax.ShapeDtypeStruct((max_len, emb_dim), table.dtype),
        scratch_shapes=[pltpu.VMEM((_NBUF, _TC_CHUNK, emb_dim), table.dtype)]
        + [pltpu.SemaphoreType.DMA] * (2 * _NBUF),
    )(table)


def _sc_copy_tail(table, start, rows):
    max_len, emb_dim = table.shape
    info = plsc.get_sparse_core_info()
    num_workers = info.num_cores * info.num_subcores
    rows_per_w = rows // num_workers
    nch = rows_per_w // _SC_CHUNK
    mesh = plsc.VectorSubcoreMesh(core_axis_name="c", subcore_axis_name="s")

    @functools.partial(
        pl.kernel,
        mesh=mesh,
        out_type=jax.ShapeDtypeStruct((rows, emb_dim), table.dtype),
        scratch_types=[
            pltpu.VMEM((2, _SC_CHUNK, emb_dim), table.dtype),
            pltpu.SemaphoreType.DMA,
            pltpu.SemaphoreType.DMA,
            pltpu.SemaphoreType.DMA,
            pltpu.SemaphoreType.DMA,
        ],
    )
    def body(table_hbm, out_hbm, buf, si0, si1, so0, so1):
        sin = (si0, si1)
        sout = (so0, so1)
        wid = lax.axis_index("s") * info.num_cores + lax.axis_index("c")
        src_base = start + wid * rows_per_w
        dst_base = wid * rows_per_w

        def cin(i):
            return pltpu.make_async_copy(
                table_hbm.at[pl.ds(src_base + i * _SC_CHUNK, _SC_CHUNK)],
                buf.at[i % 2],
                sin[i % 2],
            )

        def cout(i):
            return pltpu.make_async_copy(
                buf.at[i % 2],
                out_hbm.at[pl.ds(dst_base + i * _SC_CHUNK, _SC_CHUNK)],
                sout[i % 2],
            )

        cin(0).start()
        for i in range(nch):
            if i + 1 < nch:
                if i >= 1:
                    cout(i - 1).wait()
                cin(i + 1).start()
            cin(i).wait()
            cout(i).start()
        if nch >= 2:
            cout(nch - 2).wait()
        cout(nch - 1).wait()

    return body(table)


def kernel(x, table):
    del x
    max_len, emb_dim = table.shape
    head = max_len - _SC_ROWS
    tc_full = _tc_copy_head(table, head)
    sc_tail = _sc_copy_tail(table, head, _SC_ROWS)
    return lax.dynamic_update_slice(tc_full, sc_tail, (head, 0))[None]


# speedup vs baseline: 2.1794x; 2.1794x over previous
import jax
import jax.numpy as jnp
from jax.experimental import pallas as pl
from jax.experimental.pallas import tpu as pltpu

_CHUNK_ROWS = 2048
_NBUF = 4
_PREFETCH = 3


def kernel(x, table):
    del x
    max_len, emb_dim = table.shape
    nch = max_len // _CHUNK_ROWS

    def body(in_hbm, out_hbm, buf, *sems):
        sin = sems[:_NBUF]
        sout = sems[_NBUF:]

        def cin(i):
            return pltpu.make_async_copy(
                in_hbm.at[pl.ds(i * _CHUNK_ROWS, _CHUNK_ROWS)],
                buf.at[i % _NBUF],
                sin[i % _NBUF],
            )

        def cout(i):
            return pltpu.make_async_copy(
                buf.at[i % _NBUF],
                out_hbm.at[pl.ds(i * _CHUNK_ROWS, _CHUNK_ROWS)],
                sout[i % _NBUF],
            )

        for i in range(min(_PREFETCH, nch)):
            cin(i).start()
        for i in range(nch):
            cin(i).wait()
            cout(i).start()
            j = i + _PREFETCH
            if j < nch:
                if j >= _NBUF:
                    cout(j - _NBUF).wait()
                cin(j).start()
        for i in range(max(nch - _NBUF, 0), nch):
            cout(i).wait()

    out = pl.pallas_call(
        body,
        in_specs=[pl.BlockSpec(memory_space=pltpu.MemorySpace.HBM)],
        out_specs=pl.BlockSpec(memory_space=pltpu.MemorySpace.HBM),
        out_shape=jax.ShapeDtypeStruct((max_len, emb_dim), table.dtype),
        scratch_shapes=[pltpu.VMEM((_NBUF, _CHUNK_ROWS, emb_dim), table.dtype)]
        + [pltpu.SemaphoreType.DMA] * (2 * _NBUF),
    )(table)
    return out[None]
